# baseline (speedup 1.0000x reference)
import functools

import jax
import jax.numpy as jnp
from jax import lax
from jax.experimental import pallas as pl
from jax.experimental.pallas import tpu as pltpu
from jax.experimental.pallas import tpu_sc as plsc

B = 16384
F = 64
NC = 2
NS = 16
NW = NC * NS
BPW = B // NW
L = 16
CHUNK = 128
NCHUNK = BPW // CHUNK
GROUPS = BPW // L


def _body(users_h, items_h, ub_h, ib_h, ue_h, ie_h, out_h,
          uidx, iidx, ue_rows, ie_rows, ubv, ibv, outv, sem):
    c = lax.axis_index("c")
    s = lax.axis_index("s")
    wid = s * NC + c
    base = wid * BPW

    for j in range(NCHUNK):
        pltpu.sync_copy(users_h.at[pl.ds(base + j * CHUNK, CHUNK)], uidx.at[j])
        pltpu.sync_copy(items_h.at[pl.ds(base + j * CHUNK, CHUNK)], iidx.at[j])

    copies = []
    for j in range(NCHUNK):
        sl = pl.ds(j * CHUNK, CHUNK)
        copies.append(pltpu.make_async_copy(ue_h.at[uidx.at[j]], ue_rows.at[sl], sem))
        copies.append(pltpu.make_async_copy(ie_h.at[iidx.at[j]], ie_rows.at[sl], sem))
        copies.append(pltpu.make_async_copy(ub_h.at[uidx.at[j]], ubv.at[sl], sem))
        copies.append(pltpu.make_async_copy(ib_h.at[iidx.at[j]], ibv.at[sl], sem))
    for cp in copies:
        cp.start()
    for cp in copies:
        cp.wait()

    lane = lax.iota(jnp.int32, L)
    _dnums = lax.GatherDimensionNumbers(
        offset_dims=(), collapsed_slice_dims=(0,), start_index_map=(0,))

    def perm(x, idx):
        return lax.gather(x, idx[:, None], _dnums, (1,),
                          mode=lax.GatherScatterMode.PROMISE_IN_BOUNDS)

    def group(g, carry):
        acc = ubv[pl.ds(g * L, L)] + ibv[pl.ds(g * L, L)]
        for r in range(L):
            b = g * L + r
            p = None
            for cc in range(F // L):
                u = ue_rows[b, pl.ds(cc * L, L)]
                v = ie_rows[b, pl.ds(cc * L, L)]
                p = u * v if p is None else p + u * v
            for sh in (8, 4, 2, 1):
                p = p + perm(p, lane ^ sh)
            acc = acc + jnp.where(lane == r, p, 0.0)
        outv[pl.ds(g * L, L)] = acc
        return carry

    lax.fori_loop(0, GROUPS, group, 0)
    pltpu.sync_copy(outv, out_h.at[pl.ds(base, BPW)])


@functools.partial(jax.jit, static_argnums=())
def _sc_lfm(users, items, ub, ib, ue, ie):
    mesh = plsc.VectorSubcoreMesh(core_axis_name="c", subcore_axis_name="s")
    return pl.kernel(
        _body,
        out_type=jax.ShapeDtypeStruct((B,), jnp.float32),
        mesh=mesh,
        compiler_params=pltpu.CompilerParams(use_tc_tiling_on_sc=False),
        scratch_types=[
            pltpu.VMEM((NCHUNK, CHUNK), jnp.int32),
            pltpu.VMEM((NCHUNK, CHUNK), jnp.int32),
            pltpu.VMEM((BPW, F), jnp.float32),
            pltpu.VMEM((BPW, F), jnp.float32),
            pltpu.VMEM((BPW,), jnp.float32),
            pltpu.VMEM((BPW,), jnp.float32),
            pltpu.VMEM((BPW,), jnp.float32),
            pltpu.SemaphoreType.DMA,
        ],
    )(users, items, ub, ib, ue, ie)


def kernel(users, items, user_embeddings, item_embeddings, user_biases, item_biases):
    users = users.astype(jnp.int32)
    items = items.astype(jnp.int32)
    ub = user_biases[:, 0]
    ib = item_biases[:, 0]
    return _sc_lfm(users, items, ub, ib, user_embeddings, item_embeddings)


# speedup vs baseline: 1.3723x; 1.3723x over previous
import functools

import jax
import jax.numpy as jnp
from jax import lax
from jax.experimental import pallas as pl
from jax.experimental.pallas import tpu as pltpu
from jax.experimental.pallas import tpu_sc as plsc

B = 16384
F = 64
NC = 2
NS = 16
NW = NC * NS
BPW = B // NW
L = 16
CHUNK = 128
NCHUNK = BPW // CHUNK
GROUPS = BPW // L
D = 8


def _body(users_h, items_h, ub_h, ib_h, ue_h, ie_h, out_h,
          uidx, iidx, ue_ring, ie_ring, ubv, ibv, outv, sem, bsem):
    c = lax.axis_index("c")
    s = lax.axis_index("s")
    wid = s * NC + c
    base = wid * BPW

    pltpu.sync_copy(users_h.at[pl.ds(base, BPW)], uidx)
    pltpu.sync_copy(items_h.at[pl.ds(base, BPW)], iidx)

    bias_copies = []
    for j in range(NCHUNK):
        sl = pl.ds(j * CHUNK, CHUNK)
        bias_copies.append(pltpu.make_async_copy(ub_h.at[uidx.at[sl]], ubv.at[sl], bsem))
        bias_copies.append(pltpu.make_async_copy(ib_h.at[iidx.at[sl]], ibv.at[sl], bsem))
    for cp in bias_copies:
        cp.start()

    def issue_group(g, slot):
        iv_u = uidx[pl.ds(g * L, L)]
        iv_i = iidx[pl.ds(g * L, L)]
        for r in range(L):
            row = slot * L + r
            pltpu.make_async_copy(ue_h.at[iv_u[r]], ue_ring.at[row], sem.at[slot]).start()
            pltpu.make_async_copy(ie_h.at[iv_i[r]], ie_ring.at[row], sem.at[slot]).start()

    for g in range(D):
        issue_group(g, g)

    for cp in bias_copies:
        cp.wait()

    lane = lax.iota(jnp.int32, L)
    _dnums = lax.GatherDimensionNumbers(
        offset_dims=(), collapsed_slice_dims=(0,), start_index_map=(0,))

    def perm(x, idx):
        return lax.gather(x, idx[:, None], _dnums, (1,),
                          mode=lax.GatherScatterMode.PROMISE_IN_BOUNDS)

    def main(g, carry):
        slot = lax.rem(g, D)
        dsl = pl.ds(slot * L, L)
        pltpu.make_async_copy(ue_h.at[pl.ds(0, L)], ue_ring.at[dsl], sem.at[slot]).wait()
        pltpu.make_async_copy(ue_h.at[pl.ds(0, L)], ie_ring.at[dsl], sem.at[slot]).wait()

        acc = ubv[pl.ds(g * L, L)] + ibv[pl.ds(g * L, L)]
        for r in range(L):
            row = slot * L + r
            p = None
            for cc in range(F // L):
                u = ue_ring[row, pl.ds(cc * L, L)]
                v = ie_ring[row, pl.ds(cc * L, L)]
                p = u * v if p is None else p + u * v
            for sh in (8, 4, 2, 1):
                p = p + perm(p, lane ^ sh)
            acc = acc + jnp.where(lane == r, p, 0.0)
        outv[pl.ds(g * L, L)] = acc

        @pl.when(g + D < GROUPS)
        def _():
            issue_group(g + D, slot)

        return carry

    lax.fori_loop(0, GROUPS, main, 0)
    pltpu.sync_copy(outv, out_h.at[pl.ds(base, BPW)])


@jax.jit
def _sc_lfm(users, items, ub, ib, ue, ie):
    mesh = plsc.VectorSubcoreMesh(core_axis_name="c", subcore_axis_name="s")
    return pl.kernel(
        _body,
        out_type=jax.ShapeDtypeStruct((B,), jnp.float32),
        mesh=mesh,
        compiler_params=pltpu.CompilerParams(use_tc_tiling_on_sc=True),
        scratch_types=[
            pltpu.VMEM((BPW,), jnp.int32),
            pltpu.VMEM((BPW,), jnp.int32),
            pltpu.VMEM((D * L, F), jnp.float32),
            pltpu.VMEM((D * L, F), jnp.float32),
            pltpu.VMEM((BPW,), jnp.float32),
            pltpu.VMEM((BPW,), jnp.float32),
            pltpu.VMEM((BPW,), jnp.float32),
            pltpu.SemaphoreType.DMA((D,)),
            pltpu.SemaphoreType.DMA,
        ],
    )(users, items, ub, ib, ue, ie)


def kernel(users, items, user_embeddings, item_embeddings, user_biases, item_biases):
    users = users.astype(jnp.int32)
    items = items.astype(jnp.int32)
    ub = user_biases[:, 0]
    ib = item_biases[:, 0]
    return _sc_lfm(users, items, ub, ib, user_embeddings, item_embeddings)
